# baseline (speedup 1.0000x reference)
import jax
import jax.numpy as jnp
from jax.experimental import pallas as pl
from jax.experimental.pallas import tpu as pltpu

G = 8
TG = 1024


def _mlp_packed_kernel(x_ref, w1_ref, b1_ref, w2_ref, b2_ref, w3_ref,
                       b3_ref, w4_ref, b4_ref, o_ref):
    seg = w1_ref.shape[0]
    xraw = x_ref[...]
    tg = x_ref.shape[0] // 15
    xall = xraw.reshape(tg, 15 * 128).astype(jnp.bfloat16)
    outs = []
    for s in range(G):
        h = xall[:, seg * s:seg * (s + 1)]
        h = jnp.dot(h, w1_ref[...], preferred_element_type=jnp.float32)
        h = jnp.maximum(h + b1_ref[...], 0.0).astype(jnp.bfloat16)
        h = jnp.dot(h, w2_ref[...], preferred_element_type=jnp.float32)
        h = jnp.maximum(h + b2_ref[...], 0.0).astype(jnp.bfloat16)
        h = jnp.dot(h, w3_ref[...], preferred_element_type=jnp.float32)
        h = jnp.maximum(h + b3_ref[...], 0.0).astype(jnp.bfloat16)
        h = jnp.dot(h, w4_ref[...], preferred_element_type=jnp.float32)
        outs.append(h)
    o_ref[...] = jax.nn.sigmoid(jnp.concatenate(outs, axis=1) + b4_ref[...])


def kernel(x, w1, b1, w2, b2, w3, b3, w4, b4):
    B, f_in = x.shape
    n_out = w4.shape[1]
    lanes = 1920
    rows_per_group = lanes // f_in
    n_groups = B // rows_per_group
    seg = lanes // G

    vpg = lanes // 128
    xf = x.reshape(n_groups * vpg, 128)

    eye = jnp.eye(G, dtype=jnp.float32)

    def pack_w(w):
        return jnp.kron(eye, w).astype(jnp.bfloat16)

    def pack_b(b):
        return jnp.tile(b, (1, G))

    w1p, b1p = pack_w(w1), pack_b(b1)
    w2p, b2p = pack_w(w2), pack_b(b2)
    w3p, b3p = pack_w(w3), pack_b(b3)
    w4p = pack_w(w4)

    tg = min(TG, n_groups)
    nb = n_groups // tg

    def const(arr):
        return pl.BlockSpec(arr.shape, lambda b: (0,) * arr.ndim)

    o2 = pl.pallas_call(
        _mlp_packed_kernel,
        out_shape=jax.ShapeDtypeStruct((n_groups, rows_per_group * n_out),
                                       jnp.float32),
        grid=(nb,),
        in_specs=[pl.BlockSpec((tg * vpg, 128), lambda b: (b, 0)),
                  const(w1p), const(b1p),
                  const(w2p), const(b2p),
                  const(w3p), const(b3p),
                  const(w4p), const(b4)],
        out_specs=pl.BlockSpec((tg, rows_per_group * n_out), lambda b: (b, 0)),
        compiler_params=pltpu.CompilerParams(
            dimension_semantics=("parallel",),
            vmem_limit_bytes=48 * 1024 * 1024,
        ),
    )(xf, w1p, b1p, w2p, b2p, w3p, b3p, w4p, b4)

    return o2.reshape(B, n_out)


# speedup vs baseline: 9.1834x; 9.1834x over previous
import jax
import jax.numpy as jnp
from jax.experimental import pallas as pl
from jax.experimental.pallas import tpu as pltpu

TBN = 32768


def _mlp_t_kernel(x_ref, w1t_ref, b1t_ref, w2t_ref, b2t_ref, w3t_ref,
                  b3t_ref, w4t_ref, b4_ref, o_ref):
    a = jnp.dot(w1t_ref[...], x_ref[...], preferred_element_type=jnp.float32)
    a = jnp.maximum(a + b1t_ref[...], 0.0)
    a = jnp.dot(w2t_ref[...], a, preferred_element_type=jnp.float32)
    a = jnp.maximum(a + b2t_ref[...], 0.0)
    a = jnp.dot(w3t_ref[...], a, preferred_element_type=jnp.float32)
    a = jnp.maximum(a + b3t_ref[...], 0.0)
    a = jnp.dot(w4t_ref[...], a, preferred_element_type=jnp.float32)
    o_ref[...] = jax.nn.sigmoid(a + b4_ref[...])


def kernel(x, w1, b1, w2, b2, w3, b3, w4, b4):
    B, f_in = x.shape
    n_out = w4.shape[1]

    xt = x.T
    w1t, b1t = w1.T, b1.T
    w2t, b2t = w2.T, b2.T
    w3t, b3t = w3.T, b3.T
    w4t = w4.T

    tbn = min(TBN, B)
    n_blocks = pl.cdiv(B, tbn)

    def const(arr):
        return pl.BlockSpec(arr.shape, lambda i: (0,) * arr.ndim)

    ot = pl.pallas_call(
        _mlp_t_kernel,
        out_shape=jax.ShapeDtypeStruct((n_out, B), jnp.float32),
        grid=(n_blocks,),
        in_specs=[pl.BlockSpec((f_in, tbn), lambda i: (0, i)),
                  const(w1t), const(b1t),
                  const(w2t), const(b2t),
                  const(w3t), const(b3t),
                  const(w4t), const(b4)],
        out_specs=pl.BlockSpec((n_out, tbn), lambda i: (0, i)),
        compiler_params=pltpu.CompilerParams(
            dimension_semantics=("parallel",),
            vmem_limit_bytes=48 * 1024 * 1024,
        ),
    )(xt, w1t, b1t, w2t, b2t, w3t, b3t, w4t, b4)

    return ot.reshape(B, n_out)
